# baseline (speedup 1.0000x reference)
import functools
import math

import jax
import jax.numpy as jnp
from jax import lax
from jax.experimental import pallas as pl
from jax.experimental.pallas import tpu as pltpu
from jax.experimental.pallas import tpu_sc as plsc

D_MODEL = 64
SCALE = math.sqrt(D_MODEL)

_info = plsc.get_sparse_core_info()
_NC = _info.num_cores
_NS = _info.num_subcores
_L = _info.num_lanes
_NW = _NC * _NS

_NG = 4
_NO = 2
_C = 128
_BV = 2048


def _prep(lutT):
    D, V = lutT.shape

    def body(x_ref, o_ref):
        t = jnp.swapaxes(x_ref[...], 0, 1)
        o_ref[...] = jnp.concatenate([t, t], axis=1)

    return pl.pallas_call(
        body,
        grid=((V + _BV - 1) // _BV,),
        in_specs=[pl.BlockSpec((D, _BV), lambda j: (0, j))],
        out_specs=pl.BlockSpec((_BV, 2 * D), lambda j: (j, 0)),
        out_shape=jax.ShapeDtypeStruct((V, 2 * D), jnp.float32),
    )(lutT)


@jax.jit
def _embed(xT, table):
    T, NB = xT.shape
    n_chunks = T
    mesh = plsc.VectorSubcoreMesh(core_axis_name="c", subcore_axis_name="s")

    @functools.partial(
        pl.kernel,
        mesh=mesh,
        out_type=jax.ShapeDtypeStruct((T * NB // 2, 2 * D_MODEL), jnp.float32),
        scratch_types=(
            [pltpu.VMEM((T, _C), jnp.int32),
             pltpu.VMEM((_NG * _C, 2 * D_MODEL), jnp.float32),
             pltpu.VMEM((_NO * (_C // 2), 2 * D_MODEL), jnp.float32),
             pltpu.SemaphoreType.DMA((_NG,)),
             pltpu.SemaphoreType.DMA((_NO,))]
        ),
        compiler_params=pltpu.CompilerParams(needs_layout_passes=False),
    )
    def k(xT_hbm, table_hbm, out_hbm, idx_all, pairs, outb, gsem, wsem):
        wid = lax.axis_index("s") * _NC + lax.axis_index("c")
        b_base = wid * _C
        rowoff = wid * (_C // 2)

        pltpu.sync_copy(xT_hbm.at[:, pl.ds(b_base, _C)], idx_all)

        def start_gather(t, gb):
            pltpu.async_copy(
                table_hbm.at[idx_all.at[t]],
                pairs.at[pl.ds(gb * _C, _C)], gsem.at[gb])

        def wait_gather(gb):
            pltpu.make_async_copy(
                table_hbm.at[pl.ds(0, _C)],
                pairs.at[pl.ds(gb * _C, _C)], gsem.at[gb]).wait()

        def start_wb(t, ob):
            pltpu.async_copy(
                outb.at[pl.ds(ob * (_C // 2), _C // 2)],
                out_hbm.at[pl.ds(t * (NB // 2) + rowoff, _C // 2), :],
                wsem.at[ob])

        def wait_wb(ob):
            pltpu.make_async_copy(
                outb.at[pl.ds(ob * (_C // 2), _C // 2)],
                out_hbm.at[pl.ds(rowoff, _C // 2), :],
                wsem.at[ob]).wait()

        def produce(gb, ob):
            def rbody(r, carry):
                srca = gb * _C + r
                srcb = gb * _C + (_C // 2) + r
                dst = ob * (_C // 2) + r
                for db in range(D_MODEL // _L):
                    sl = pl.ds(db * _L, _L)
                    outb[dst, sl] = pairs[srca, sl] * SCALE
                    outb[dst, pl.ds(D_MODEL + db * _L, _L)] = (
                        pairs[srcb, sl] * SCALE)
                return carry

            lax.fori_loop(0, _C // 2, rbody, 0, unroll=4)

        def prime(b, carry):
            start_gather(b, b)
            return carry

        lax.fori_loop(0, _NG, prime, 0)

        def step(t, carry):
            gb = lax.bitwise_and(t, _NG - 1)
            ob = lax.bitwise_and(t, _NO - 1)
            wait_gather(gb)

            @pl.when(t >= _NO)
            def _():
                wait_wb(ob)

            produce(gb, ob)

            @pl.when(t + _NG < n_chunks)
            def _():
                start_gather(t + _NG, gb)

            start_wb(t, ob)
            return carry

        lax.fori_loop(0, n_chunks, step, 0)

        for ob in range(_NO):
            wait_wb(ob)

    return k(xT, table)


def _post(out_lin, T, NB):
    def body(x_ref, o_ref):
        blk = x_ref[...]
        pieces = []
        for i in range(4):
            sub = blk[i * 64:(i + 1) * 64, :]
            pieces.append(jnp.swapaxes(sub[:, :D_MODEL], 0, 1))
            pieces.append(jnp.swapaxes(sub[:, D_MODEL:], 0, 1))
        o_ref[0] = jnp.concatenate(pieces, axis=1)

    return pl.pallas_call(
        body,
        grid=(T, NB // 512),
        in_specs=[pl.BlockSpec((256, 2 * D_MODEL),
                               lambda t, cb: (t * 8 + cb, 0))],
        out_specs=pl.BlockSpec((1, D_MODEL, 512), lambda t, cb: (t, 0, cb)),
        out_shape=jax.ShapeDtypeStruct((T, D_MODEL, NB), jnp.float32),
    )(out_lin)


def kernel(x, lut):
    T, NB = x.shape[1], x.shape[0]
    table = _prep(lut.T)
    out_lin = _embed(x.T, table)
    out_p = _post(out_lin, T, NB)
    return jnp.transpose(out_p, (2, 0, 1))


# speedup vs baseline: 1.0262x; 1.0262x over previous
import functools
import math

import jax
import jax.numpy as jnp
from jax import lax
from jax.experimental import pallas as pl
from jax.experimental.pallas import tpu as pltpu
from jax.experimental.pallas import tpu_sc as plsc

D_MODEL = 64
SCALE = math.sqrt(D_MODEL)

_info = plsc.get_sparse_core_info()
_NC = _info.num_cores
_NS = _info.num_subcores
_L = _info.num_lanes
_NW = _NC * _NS

_NG = 4
_NO = 2
_C = 128
_BV = 2048


def _prep(lutT):
    D, V = lutT.shape

    def body(x_ref, o_ref):
        t = jnp.swapaxes(x_ref[...], 0, 1)
        o_ref[...] = jnp.concatenate([t, t], axis=1)

    return pl.pallas_call(
        body,
        grid=((V + _BV - 1) // _BV,),
        in_specs=[pl.BlockSpec((D, _BV), lambda j: (0, j))],
        out_specs=pl.BlockSpec((_BV, 2 * D), lambda j: (j, 0)),
        out_shape=jax.ShapeDtypeStruct((V, 2 * D), jnp.float32),
    )(lutT)


@jax.jit
def _embed(xT, table):
    T, NB = xT.shape
    n_chunks = T
    mesh = plsc.VectorSubcoreMesh(core_axis_name="c", subcore_axis_name="s")

    @functools.partial(
        pl.kernel,
        mesh=mesh,
        out_type=jax.ShapeDtypeStruct((T * NB // 2, 2 * D_MODEL), jnp.float32),
        scratch_types=(
            [pltpu.VMEM((T, _C), jnp.int32),
             pltpu.VMEM((_NG * _C, 2 * D_MODEL), jnp.float32),
             pltpu.VMEM((_NO * (_C // 2), 2 * D_MODEL), jnp.float32),
             pltpu.SemaphoreType.DMA((_NG,)),
             pltpu.SemaphoreType.DMA((_NO,))]
        ),
        compiler_params=pltpu.CompilerParams(needs_layout_passes=False),
    )
    def k(xT_hbm, table_hbm, out_hbm, idx_all, pairs, outb, gsem, wsem):
        wid = lax.axis_index("s") * _NC + lax.axis_index("c")
        b_base = wid * _C
        rowoff = wid * (_C // 2)

        pltpu.sync_copy(xT_hbm.at[:, pl.ds(b_base, _C)], idx_all)

        def start_gather(t, gb):
            pltpu.async_copy(
                table_hbm.at[idx_all.at[t]],
                pairs.at[pl.ds(gb * _C, _C)], gsem.at[gb])

        def wait_gather(gb):
            pltpu.make_async_copy(
                table_hbm.at[pl.ds(0, _C)],
                pairs.at[pl.ds(gb * _C, _C)], gsem.at[gb]).wait()

        def start_wb(t, ob):
            pltpu.async_copy(
                outb.at[pl.ds(ob * (_C // 2), _C // 2)],
                out_hbm.at[pl.ds(t * (NB // 2) + rowoff, _C // 2), :],
                wsem.at[ob])

        def wait_wb(ob):
            pltpu.make_async_copy(
                outb.at[pl.ds(ob * (_C // 2), _C // 2)],
                out_hbm.at[pl.ds(rowoff, _C // 2), :],
                wsem.at[ob]).wait()

        def produce(gb, ob):
            def rbody(r, carry):
                srca = gb * _C + r
                srcb = gb * _C + (_C // 2) + r
                dst = ob * (_C // 2) + r
                for db in range(D_MODEL // _L):
                    sl = pl.ds(db * _L, _L)
                    outb[dst, sl] = pairs[srca, sl] * SCALE
                    outb[dst, pl.ds(D_MODEL + db * _L, _L)] = (
                        pairs[srcb, sl] * SCALE)
                return carry

            lax.fori_loop(0, _C // 2, rbody, 0, unroll=4)

        def prime(b, carry):
            start_gather(b, b)
            return carry

        lax.fori_loop(0, _NG, prime, 0)

        def step(t, carry):
            gb = lax.bitwise_and(t, _NG - 1)
            ob = lax.bitwise_and(t, _NO - 1)
            wait_gather(gb)

            @pl.when(t >= _NO)
            def _():
                wait_wb(ob)

            produce(gb, ob)

            @pl.when(t + _NG < n_chunks)
            def _():
                start_gather(t + _NG, gb)

            start_wb(t, ob)
            return carry

        lax.fori_loop(0, n_chunks, step, 0)

        for ob in range(_NO):
            wait_wb(ob)

    return k(xT, table)


def _post(out_lin, T, NB):
    def body(x_ref, o_ref):
        t = jnp.swapaxes(x_ref[...], 0, 1)
        pieces = []
        for m in range(4):
            for h in range(2):
                pieces.append(
                    t[h * 64:(h + 1) * 64, m * 64:(m + 1) * 64])
        o_ref[0] = jnp.concatenate(pieces, axis=1)

    return pl.pallas_call(
        body,
        grid=(T, NB // 512),
        in_specs=[pl.BlockSpec((256, 2 * D_MODEL),
                               lambda t, cb: (t * 8 + cb, 0))],
        out_specs=pl.BlockSpec((1, D_MODEL, 512), lambda t, cb: (t, 0, cb)),
        out_shape=jax.ShapeDtypeStruct((T, D_MODEL, NB), jnp.float32),
    )(out_lin)


def kernel(x, lut):
    T, NB = x.shape[1], x.shape[0]
    table = _prep(lut.T)
    out_lin = _embed(x.T, table)
    out_p = _post(out_lin, T, NB)
    return jnp.transpose(out_p, (2, 0, 1))


# speedup vs baseline: 1.0411x; 1.0146x over previous
import functools
import math

import jax
import jax.numpy as jnp
from jax import lax
from jax.experimental import pallas as pl
from jax.experimental.pallas import tpu as pltpu
from jax.experimental.pallas import tpu_sc as plsc

D_MODEL = 64
SCALE = math.sqrt(D_MODEL)

_info = plsc.get_sparse_core_info()
_NC = _info.num_cores
_NS = _info.num_subcores
_L = _info.num_lanes
_NW = _NC * _NS

_NG = 4
_NO = 2
_C = 128
_BV = 2048


def _prep(lutT):
    D, V = lutT.shape

    def body(x_ref, o_ref):
        t = jnp.swapaxes(x_ref[...], 0, 1)
        o_ref[...] = jnp.concatenate([t, t], axis=1)

    return pl.pallas_call(
        body,
        grid=((V + _BV - 1) // _BV,),
        in_specs=[pl.BlockSpec((D, _BV), lambda j: (0, j))],
        out_specs=pl.BlockSpec((_BV, 2 * D), lambda j: (j, 0)),
        out_shape=jax.ShapeDtypeStruct((V, 2 * D), jnp.float32),
    )(lutT)


@jax.jit
def _embed(xT, table):
    T, NB = xT.shape
    n_chunks = T
    mesh = plsc.VectorSubcoreMesh(core_axis_name="c", subcore_axis_name="s")

    @functools.partial(
        pl.kernel,
        mesh=mesh,
        out_type=jax.ShapeDtypeStruct((T * NB // 2, 2 * D_MODEL), jnp.float32),
        scratch_types=(
            [pltpu.VMEM((T, _C), jnp.int32),
             pltpu.VMEM((_NG * _C, 2 * D_MODEL), jnp.float32),
             pltpu.VMEM((_NO * (_C // 2), 2 * D_MODEL), jnp.float32),
             pltpu.SemaphoreType.DMA((_NG,)),
             pltpu.SemaphoreType.DMA((_NO,))]
        ),
        compiler_params=pltpu.CompilerParams(needs_layout_passes=False),
    )
    def k(xT_hbm, table_hbm, out_hbm, idx_all, pairs, outb, gsem, wsem):
        wid = lax.axis_index("s") * _NC + lax.axis_index("c")
        b_base = wid * _C
        rowoff = wid * (_C // 2)

        pltpu.sync_copy(xT_hbm.at[:, pl.ds(b_base, _C)], idx_all)

        def start_gather(t, gb):
            pltpu.async_copy(
                table_hbm.at[idx_all.at[t]],
                pairs.at[pl.ds(gb * _C, _C)], gsem.at[gb])

        def wait_gather(gb):
            pltpu.make_async_copy(
                table_hbm.at[pl.ds(0, _C)],
                pairs.at[pl.ds(gb * _C, _C)], gsem.at[gb]).wait()

        def start_wb(t, ob):
            pltpu.async_copy(
                outb.at[pl.ds(ob * (_C // 2), _C // 2)],
                out_hbm.at[pl.ds(t * (NB // 2) + rowoff, _C // 2), :],
                wsem.at[ob])

        def wait_wb(ob):
            pltpu.make_async_copy(
                outb.at[pl.ds(ob * (_C // 2), _C // 2)],
                out_hbm.at[pl.ds(rowoff, _C // 2), :],
                wsem.at[ob]).wait()

        def produce(gb, ob):
            def rbody(r, carry):
                srca = gb * _C + r
                srcb = gb * _C + (_C // 2) + r
                dst = ob * (_C // 2) + r
                for db in range(D_MODEL // _L):
                    sl = pl.ds(db * _L, _L)
                    outb[dst, sl] = pairs[srca, sl] * SCALE
                    outb[dst, pl.ds(D_MODEL + db * _L, _L)] = (
                        pairs[srcb, sl] * SCALE)
                return carry

            lax.fori_loop(0, _C // 2, rbody, 0, unroll=4)

        def prime(b, carry):
            start_gather(b, b)
            return carry

        lax.fori_loop(0, _NG, prime, 0)

        def step(t, carry):
            gb = lax.bitwise_and(t, _NG - 1)
            ob = lax.bitwise_and(t, _NO - 1)
            wait_gather(gb)

            @pl.when(t >= _NO)
            def _():
                wait_wb(ob)

            produce(gb, ob)

            @pl.when(t + _NG < n_chunks)
            def _():
                start_gather(t + _NG, gb)

            start_wb(t, ob)
            return carry

        lax.fori_loop(0, n_chunks, step, 0)

        for ob in range(_NO):
            wait_wb(ob)

    return k(xT, table)


def _post(out_lin, T, NB):
    def body(x_ref, o_ref):
        t = jnp.swapaxes(x_ref[...], 0, 1)
        pieces = []
        for m in range(4):
            for h in range(2):
                pieces.append(
                    t[h * 64:(h + 1) * 64, m * 64:(m + 1) * 64])
        o_ref[0] = jnp.concatenate(pieces, axis=1)

    return pl.pallas_call(
        body,
        grid=(T, NB // 512),
        in_specs=[pl.BlockSpec((256, 2 * D_MODEL),
                               lambda t, cb: (t * 8 + cb, 0))],
        out_specs=pl.BlockSpec((1, D_MODEL, 512), lambda t, cb: (t, 0, cb)),
        out_shape=jax.ShapeDtypeStruct((T, D_MODEL, NB), jnp.float32),
    )(out_lin)


def kernel(x, lut):
    T, NB = x.shape[1], x.shape[0]
    table = _prep(lut.T)
    out_lin = _embed(x.T, table)
    out6 = out_lin.reshape(T, NB // _C, _C // 2, 2, D_MODEL)
    return out6.transpose(1, 3, 2, 0, 4).reshape(NB, T, D_MODEL)


# speedup vs baseline: 1.4799x; 1.4215x over previous
import functools
import math

import jax
import jax.numpy as jnp
from jax import lax
from jax.experimental import pallas as pl
from jax.experimental.pallas import tpu as pltpu
from jax.experimental.pallas import tpu_sc as plsc

D_MODEL = 64
SCALE = math.sqrt(D_MODEL)

_info = plsc.get_sparse_core_info()
_NC = _info.num_cores
_NS = _info.num_subcores
_L = _info.num_lanes
_NW = _NC * _NS

_NBUF = 4
_WBUF = 2


@functools.partial(jax.jit, static_argnums=(2, 3))
def _gather_scale(idx_flat, lut, B, C):
    b_per_w = B // _NW
    n_chunks = b_per_w // C
    assert n_chunks % _NBUF == 0 and n_chunks >= 2 * _NBUF
    mesh = plsc.VectorSubcoreMesh(core_axis_name="c", subcore_axis_name="s")

    @functools.partial(
        pl.kernel,
        mesh=mesh,
        out_type=jax.ShapeDtypeStruct((B, D_MODEL), jnp.float32),
        scratch_types=(
            [pltpu.VMEM((b_per_w,), jnp.int32),
             pltpu.VMEM((_NBUF, C, D_MODEL), jnp.float32),
             pltpu.VMEM((_WBUF, C, D_MODEL), jnp.float32)]
            + [pltpu.SemaphoreType.DMA] * (_NBUF + _WBUF)
        ),
        compiler_params=pltpu.CompilerParams(use_tc_tiling_on_sc=False),
    )
    def k(idx_hbm, table_hbm, out_hbm, idx_all, grows, wrows, *sems):
        gsems = sems[:_NBUF]
        wsems = sems[_NBUF:]
        wid = lax.axis_index("s") * _NC + lax.axis_index("c")
        base = wid * b_per_w
        pltpu.sync_copy(idx_hbm.at[pl.ds(base, b_per_w)], idx_all)

        def start_gather(i_chunk, gb):
            pltpu.async_copy(
                table_hbm.at[idx_all.at[pl.ds(i_chunk * C, C)]],
                grows.at[gb], gsems[gb])

        def wait_gather(gb):
            pltpu.make_async_copy(
                table_hbm.at[pl.ds(0, C)], grows.at[gb], gsems[gb]).wait()

        def start_wb(i_chunk, wb):
            pltpu.async_copy(
                wrows.at[wb], out_hbm.at[pl.ds(base + i_chunk * C, C)],
                wsems[wb])

        def wait_wb(wb):
            pltpu.make_async_copy(
                wrows.at[wb], out_hbm.at[pl.ds(base, C)], wsems[wb]).wait()

        def scale(gb, wb):
            def row_body(r, carry):
                for j in range(D_MODEL // _L):
                    sl = pl.ds(j * _L, _L)
                    wrows[wb, r, sl] = grows[gb, r, sl] * SCALE
                return carry
            lax.fori_loop(0, C, row_body, 0, unroll=4)

        for b in range(_NBUF):
            start_gather(b, b)

        for b in range(_NBUF):
            wait_gather(b)
            if b >= _WBUF:
                wait_wb(b % _WBUF)
            scale(b, b % _WBUF)
            start_gather(b + _NBUF, b)
            start_wb(b, b % _WBUF)

        def outer(g, carry):
            for b in range(_NBUF):
                i = g * _NBUF + b
                wait_gather(b)
                wait_wb(b % _WBUF)
                scale(b, b % _WBUF)
                start_gather(i + _NBUF, b)
                start_wb(i, b % _WBUF)
            return carry

        lax.fori_loop(1, n_chunks // _NBUF - 1, outer, 0)

        for b in range(_NBUF):
            i = n_chunks - _NBUF + b
            wait_gather(b)
            wait_wb(b % _WBUF)
            scale(b, b % _WBUF)
            start_wb(i, b % _WBUF)

        for wb in range(_WBUF):
            wait_wb(wb)

    return k(idx_flat, lut)


def kernel(x, lut):
    B = x.shape[0] * x.shape[1]
    idx_flat = x.reshape(B).astype(jnp.int32)
    out = _gather_scale(idx_flat, lut, B, 128)
    return out.reshape(x.shape[0], x.shape[1], D_MODEL)


# speedup vs baseline: 1.5824x; 1.0692x over previous
import functools
import math

import jax
import jax.numpy as jnp
from jax import lax
from jax.experimental import pallas as pl
from jax.experimental.pallas import tpu as pltpu
from jax.experimental.pallas import tpu_sc as plsc

D_MODEL = 64
SCALE = math.sqrt(D_MODEL)

_info = plsc.get_sparse_core_info()
_NC = _info.num_cores
_NS = _info.num_subcores
_L = _info.num_lanes
_NW = _NC * _NS

_NBUF = 4
_WBUF = 2


@functools.partial(jax.jit, static_argnums=(2, 3))
def _gather_scale(idx_flat, lut, B, C):
    b_per_w = B // _NW
    n_chunks = b_per_w // C
    assert n_chunks % _NBUF == 0 and n_chunks >= 2 * _NBUF
    mesh = plsc.VectorSubcoreMesh(core_axis_name="c", subcore_axis_name="s")

    @functools.partial(
        pl.kernel,
        mesh=mesh,
        out_type=jax.ShapeDtypeStruct((B, D_MODEL), jnp.float32),
        scratch_types=(
            [pltpu.VMEM((b_per_w,), jnp.int32),
             pltpu.VMEM((_NBUF, C, D_MODEL), jnp.float32),
             pltpu.VMEM((_WBUF, C, D_MODEL), jnp.float32)]
            + [pltpu.SemaphoreType.DMA] * (_NBUF + _WBUF)
        ),
        compiler_params=pltpu.CompilerParams(use_tc_tiling_on_sc=False),
    )
    def k(idx_hbm, table_hbm, out_hbm, idx_all, grows, wrows, *sems):
        gsems = sems[:_NBUF]
        wsems = sems[_NBUF:]
        wid = lax.axis_index("s") * _NC + lax.axis_index("c")
        base = wid * b_per_w
        pltpu.sync_copy(idx_hbm.at[pl.ds(base, b_per_w)], idx_all)

        def start_gather(i_chunk, gb):
            pltpu.async_copy(
                table_hbm.at[idx_all.at[pl.ds(i_chunk * C, C)]],
                grows.at[gb], gsems[gb])

        def wait_gather(gb):
            pltpu.make_async_copy(
                table_hbm.at[pl.ds(0, C)], grows.at[gb], gsems[gb]).wait()

        def start_wb(i_chunk, wb):
            pltpu.async_copy(
                wrows.at[wb], out_hbm.at[pl.ds(base + i_chunk * C, C)],
                wsems[wb])

        def wait_wb(wb):
            pltpu.make_async_copy(
                wrows.at[wb], out_hbm.at[pl.ds(base, C)], wsems[wb]).wait()

        def scale(gb, wb):
            def row_body(r, carry):
                for j in range(D_MODEL // _L):
                    sl = pl.ds(j * _L, _L)
                    wrows[wb, r, sl] = grows[gb, r, sl] * SCALE
                return carry
            lax.fori_loop(0, C, row_body, 0, unroll=4)

        for b in range(_NBUF):
            start_gather(b, b)

        for b in range(_NBUF):
            wait_gather(b)
            if b >= _WBUF:
                wait_wb(b % _WBUF)
            scale(b, b % _WBUF)
            start_gather(b + _NBUF, b)
            start_wb(b, b % _WBUF)

        def outer(g, carry):
            for b in range(_NBUF):
                i = g * _NBUF + b
                wait_gather(b)
                wait_wb(b % _WBUF)
                scale(b, b % _WBUF)
                start_gather(i + _NBUF, b)
                start_wb(i, b % _WBUF)
            return carry

        lax.fori_loop(1, n_chunks // _NBUF - 1, outer, 0)

        for b in range(_NBUF):
            i = n_chunks - _NBUF + b
            wait_gather(b)
            wait_wb(b % _WBUF)
            scale(b, b % _WBUF)
            start_wb(i, b % _WBUF)

        for wb in range(_WBUF):
            wait_wb(wb)

    return k(idx_flat, lut)


def kernel(x, lut):
    B = x.shape[0] * x.shape[1]
    idx_flat = x.reshape(B).astype(jnp.int32)
    out = _gather_scale(idx_flat, lut, B, 256)
    return out.reshape(x.shape[0], x.shape[1], D_MODEL)
